# Initial kernel scaffold; baseline (speedup 1.0000x reference)
import jax
import jax.numpy as jnp
from jax.experimental import pallas as pl


def kernel(hydraulic_head, conduit_size, transmissivity, melt_forcing, creep_closure, length_of_link, area_at_node, node_at_link_head, node_at_link_tail, links_at_node, link_dirs_at_node, node_is_boundary):
    raise NotImplementedError("write your pallas kernel here")


# speedup vs baseline: 56.5331x; 56.5331x over previous
import functools

import jax
import jax.numpy as jnp
from jax import lax
from jax.experimental import pallas as pl
from jax.experimental.pallas import tpu as pltpu
from jax.experimental.pallas import tpu_sc as plsc

N = 100000
E = 400000
L = 8
DT = 0.1

NC = 2
NS = 16
NW = NC * NS

E_PAD = 409600
EW = E_PAD // NW
ER = EW // 128
N_PAD = 102400
NWK = N_PAD // NW
NR = NWK // 128

_mesh = plsc.VectorSubcoreMesh(core_axis_name="c", subcore_axis_name="s")


@functools.partial(
    pl.kernel,
    mesh=_mesh,
    out_type=jax.ShapeDtypeStruct((NW, EW), jnp.float32),
    scratch_types=[
        pltpu.VMEM((EW,), jnp.int32),
        pltpu.VMEM((EW,), jnp.int32),
        pltpu.VMEM((EW,), jnp.float32),
        pltpu.VMEM((EW,), jnp.float32),
        pltpu.VMEM((EW,), jnp.float32),
        pltpu.VMEM((EW,), jnp.float32),
        pltpu.VMEM((EW,), jnp.float32),
        pltpu.VMEM((EW,), jnp.float32),
        pltpu.VMEM((EW,), jnp.float32),
        pltpu.SemaphoreType.DMA,
    ],
)
def _flux_kernel(head_hbm, cs_hbm, trans_hbm, len_hbm, hidx_hbm, tidx_hbm,
                 out_hbm, hidx_v, tidx_v, hh_v, ht_v, ch_v, ct_v, tr_v,
                 ln_v, fl_v, sem):
    wid = lax.axis_index("s") * NC + lax.axis_index("c")
    pltpu.sync_copy(hidx_hbm.at[wid], hidx_v)
    pltpu.sync_copy(tidx_hbm.at[wid], tidx_v)
    pltpu.sync_copy(trans_hbm.at[wid], tr_v)
    pltpu.sync_copy(len_hbm.at[wid], ln_v)

    def fire(r, carry):
        sl = pl.ds(r * 128, 128)
        pltpu.async_copy(head_hbm.at[hidx_v.at[sl]], hh_v.at[sl], sem)
        pltpu.async_copy(head_hbm.at[tidx_v.at[sl]], ht_v.at[sl], sem)
        pltpu.async_copy(cs_hbm.at[hidx_v.at[sl]], ch_v.at[sl], sem)
        pltpu.async_copy(cs_hbm.at[tidx_v.at[sl]], ct_v.at[sl], sem)
        return carry

    lax.fori_loop(0, ER, fire, 0)
    for buf in (hh_v, ht_v, ch_v, ct_v):
        pltpu.make_async_copy(out_hbm.at[wid], buf, sem).wait()

    def body(r, carry):
        for c in range(8):
            s = pl.ds(r * 128 + c * 16, 16)
            fl_v[s] = ((-0.5) * tr_v[s] * (ch_v[s] + ct_v[s])
                       * ((hh_v[s] - ht_v[s]) / ln_v[s]))
        return carry

    lax.fori_loop(0, ER, body, 0)
    pltpu.sync_copy(fl_v, out_hbm.at[wid])


@functools.partial(
    pl.kernel,
    mesh=_mesh,
    out_type=jax.ShapeDtypeStruct((NW, NWK), jnp.float32),
    scratch_types=[
        pltpu.VMEM((L * NWK,), jnp.int32),
        pltpu.VMEM((L * NWK,), jnp.float32),
        pltpu.VMEM((L * NWK,), jnp.float32),
        pltpu.VMEM((NWK,), jnp.float32),
        pltpu.VMEM((NWK,), jnp.float32),
        pltpu.VMEM((NWK,), jnp.float32),
        pltpu.VMEM((NWK,), jnp.float32),
        pltpu.VMEM((NWK,), jnp.float32),
        pltpu.VMEM((NWK,), jnp.float32),
        pltpu.VMEM((NWK,), jnp.float32),
        pltpu.SemaphoreType.DMA,
    ],
)
def _node_kernel(flux_hbm, links_hbm, dirs_hbm, head_hbm, cs_hbm, melt_hbm,
                 creep_hbm, area_hbm, bnd_hbm, out_hbm, lidx_v, vals_v,
                 dirs_v, head_v, cs_v, melt_v, creep_v, area_v, bnd_v,
                 out_v, sem):
    wid = lax.axis_index("s") * NC + lax.axis_index("c")
    pltpu.sync_copy(links_hbm.at[wid], lidx_v)
    pltpu.sync_copy(dirs_hbm.at[wid], dirs_v)
    pltpu.sync_copy(head_hbm.at[wid], head_v)
    pltpu.sync_copy(cs_hbm.at[wid], cs_v)
    pltpu.sync_copy(melt_hbm.at[wid], melt_v)
    pltpu.sync_copy(creep_hbm.at[wid], creep_v)
    pltpu.sync_copy(area_hbm.at[wid], area_v)
    pltpu.sync_copy(bnd_hbm.at[wid], bnd_v)

    def fire(r, carry):
        for l in range(L):
            sl = pl.ds(l * NWK + r * 128, 128)
            pltpu.async_copy(flux_hbm.at[lidx_v.at[sl]], vals_v.at[sl], sem)
        return carry

    lax.fori_loop(0, NR, fire, 0)
    pltpu.make_async_copy(dirs_hbm.at[wid], vals_v, sem).wait()

    def body(r, carry):
        for c in range(8):
            off = r * 128 + c * 16
            so = pl.ds(off, 16)
            acc = dirs_v[pl.ds(off, 16)] * vals_v[pl.ds(off, 16)]
            for l in range(1, L):
                sl = pl.ds(l * NWK + off, 16)
                acc = acc + dirs_v[sl] * vals_v[sl]
            h = head_v[so]
            cs = cs_v[so]
            melt = melt_v[so]
            creep = creep_v[so]
            area = area_v[so]
            bnd = bnd_v[so]
            div = jnp.where(bnd > 0.5, h, acc / area)
            em = melt + 0.1 * div
            k1 = em - creep * cs
            k2 = em - creep * (cs + k1 * (DT / 2))
            k3 = em - creep * (cs + k2 * (DT / 2))
            k4 = em - creep * (cs + k3 * DT)
            out_v[so] = cs + (DT / 6.0) * (k1 + 2 * k2 + 2 * k3 + k4)
        return carry

    lax.fori_loop(0, NR, body, 0)
    pltpu.sync_copy(out_v, out_hbm.at[wid])


def kernel(hydraulic_head, conduit_size, transmissivity, melt_forcing,
           creep_closure, length_of_link, area_at_node,
           node_at_link_head, node_at_link_tail, links_at_node,
           link_dirs_at_node, node_is_boundary):
    ep = E_PAD - E
    hidx = jnp.pad(node_at_link_head, (0, ep)).reshape(NW, EW)
    tidx = jnp.pad(node_at_link_tail, (0, ep)).reshape(NW, EW)
    trans = jnp.pad(transmissivity, (0, ep)).reshape(NW, EW)
    lenl = jnp.pad(length_of_link, (0, ep), constant_values=1.0
                   ).reshape(NW, EW)

    flux = _flux_kernel(hydraulic_head, conduit_size, trans, lenl,
                        hidx, tidx)
    flux_flat = flux.reshape(E_PAD)

    np_ = N_PAD - N
    linksT = jnp.pad(links_at_node, ((0, np_), (0, 0))).T
    dirsT = jnp.pad(link_dirs_at_node, ((0, np_), (0, 0))).T
    links_r = linksT.reshape(L, NW, NWK).transpose(1, 0, 2).reshape(NW, L * NWK)
    dirs_r = dirsT.astype(jnp.float32).reshape(L, NW, NWK
                                               ).transpose(1, 0, 2).reshape(NW, L * NWK)
    head_r = jnp.pad(hydraulic_head, (0, np_)).reshape(NW, NWK)
    cs_r = jnp.pad(conduit_size, (0, np_)).reshape(NW, NWK)
    melt_r = jnp.pad(melt_forcing, (0, np_)).reshape(NW, NWK)
    creep_r = jnp.pad(creep_closure, (0, np_)).reshape(NW, NWK)
    area_r = jnp.pad(area_at_node, (0, np_), constant_values=1.0
                     ).reshape(NW, NWK)
    bnd_r = jnp.pad(node_is_boundary.astype(jnp.float32), (0, np_),
                    constant_values=1.0).reshape(NW, NWK)

    out = _node_kernel(flux_flat, links_r, dirs_r, head_r, cs_r, melt_r,
                       creep_r, area_r, bnd_r)
    return out.reshape(N_PAD)[:N]


# speedup vs baseline: 145.3815x; 2.5716x over previous
import functools

import jax
import jax.numpy as jnp
from jax import lax
from jax.experimental import pallas as pl
from jax.experimental.pallas import tpu as pltpu
from jax.experimental.pallas import tpu_sc as plsc

N = 100000
E = 400000
L = 8
DT = 0.1

NC = 2
NS = 16
NW = NC * NS

E_PAD = 409600
EW = E_PAD // NW
ER = EW // 128
N_PAD = 102400
NWK = N_PAD // NW
NR = NWK // 128

_mesh = plsc.VectorSubcoreMesh(core_axis_name="c", subcore_axis_name="s")


@functools.partial(
    pl.kernel,
    mesh=_mesh,
    out_type=jax.ShapeDtypeStruct((NW, EW), jnp.float32),
    scratch_types=[
        pltpu.VMEM_SHARED((N_PAD,), jnp.float32),
        pltpu.VMEM_SHARED((N_PAD,), jnp.float32),
        pltpu.VMEM((EW,), jnp.int32),
        pltpu.VMEM((EW,), jnp.int32),
        pltpu.VMEM((EW,), jnp.float32),
        pltpu.VMEM((EW,), jnp.float32),
        pltpu.VMEM((EW,), jnp.float32),
        pltpu.VMEM((EW,), jnp.float32),
        pltpu.VMEM((EW,), jnp.float32),
        pltpu.VMEM((EW,), jnp.float32),
        pltpu.VMEM((EW,), jnp.float32),
        pltpu.SemaphoreType.DMA,
    ],
)
def _flux_kernel(head_hbm, cs_hbm, trans_hbm, len_hbm, hidx_hbm, tidx_hbm,
                 out_hbm, head_sh, cs_sh, hidx_v, tidx_v, hh_v, ht_v, ch_v,
                 ct_v, tr_v, ln_v, fl_v, sem):
    sid = lax.axis_index("s")
    wid = sid * NC + lax.axis_index("c")
    seg = N_PAD // NS
    ss = pl.ds(sid * seg, seg)
    pltpu.sync_copy(head_hbm.at[ss], head_sh.at[ss])
    pltpu.sync_copy(cs_hbm.at[ss], cs_sh.at[ss])

    pltpu.sync_copy(hidx_hbm.at[wid], hidx_v)
    pltpu.sync_copy(tidx_hbm.at[wid], tidx_v)
    pltpu.sync_copy(trans_hbm.at[wid], tr_v)
    pltpu.sync_copy(len_hbm.at[wid], ln_v)
    plsc.subcore_barrier()

    def fire(r, carry):
        sl = pl.ds(r * 128, 128)
        pltpu.async_copy(head_sh.at[hidx_v.at[sl]], hh_v.at[sl], sem)
        pltpu.async_copy(head_sh.at[tidx_v.at[sl]], ht_v.at[sl], sem)
        pltpu.async_copy(cs_sh.at[hidx_v.at[sl]], ch_v.at[sl], sem)
        pltpu.async_copy(cs_sh.at[tidx_v.at[sl]], ct_v.at[sl], sem)
        return carry

    lax.fori_loop(0, ER, fire, 0)
    for buf in (hh_v, ht_v, ch_v, ct_v):
        pltpu.make_async_copy(out_hbm.at[wid], buf, sem).wait()

    def body(r, carry):
        for c in range(8):
            s = pl.ds(r * 128 + c * 16, 16)
            fl_v[s] = ((-0.5) * tr_v[s] * (ch_v[s] + ct_v[s])
                       * ((hh_v[s] - ht_v[s]) / ln_v[s]))
        return carry

    lax.fori_loop(0, ER, body, 0)
    pltpu.sync_copy(fl_v, out_hbm.at[wid])


@functools.partial(
    pl.kernel,
    mesh=_mesh,
    out_type=jax.ShapeDtypeStruct((NW, NWK), jnp.float32),
    scratch_types=[
        pltpu.VMEM_SHARED((E_PAD,), jnp.float32),
        pltpu.VMEM((L * NWK,), jnp.int32),
        pltpu.VMEM((L * NWK,), jnp.float32),
        pltpu.VMEM((L * NWK,), jnp.float32),
        pltpu.VMEM((NWK,), jnp.float32),
        pltpu.VMEM((NWK,), jnp.float32),
        pltpu.VMEM((NWK,), jnp.float32),
        pltpu.VMEM((NWK,), jnp.float32),
        pltpu.VMEM((NWK,), jnp.float32),
        pltpu.VMEM((NWK,), jnp.float32),
        pltpu.VMEM((NWK,), jnp.float32),
        pltpu.SemaphoreType.DMA,
    ],
)
def _node_kernel(flux_hbm, links_hbm, dirs_hbm, head_hbm, cs_hbm, melt_hbm,
                 creep_hbm, area_hbm, bnd_hbm, out_hbm, flux_sh, lidx_v,
                 vals_v, dirs_v, head_v, cs_v, melt_v, creep_v, area_v,
                 bnd_v, out_v, sem):
    sid = lax.axis_index("s")
    wid = sid * NC + lax.axis_index("c")
    seg = E_PAD // NS
    ss = pl.ds(sid * seg, seg)
    pltpu.sync_copy(flux_hbm.at[ss], flux_sh.at[ss])

    pltpu.sync_copy(links_hbm.at[wid], lidx_v)
    pltpu.sync_copy(dirs_hbm.at[wid], dirs_v)
    pltpu.sync_copy(head_hbm.at[wid], head_v)
    pltpu.sync_copy(cs_hbm.at[wid], cs_v)
    pltpu.sync_copy(melt_hbm.at[wid], melt_v)
    pltpu.sync_copy(creep_hbm.at[wid], creep_v)
    pltpu.sync_copy(area_hbm.at[wid], area_v)
    pltpu.sync_copy(bnd_hbm.at[wid], bnd_v)
    plsc.subcore_barrier()

    def fire(r, carry):
        for l in range(L):
            sl = pl.ds(l * NWK + r * 128, 128)
            pltpu.async_copy(flux_sh.at[lidx_v.at[sl]], vals_v.at[sl], sem)
        return carry

    lax.fori_loop(0, NR, fire, 0)
    pltpu.make_async_copy(dirs_hbm.at[wid], vals_v, sem).wait()

    def body(r, carry):
        for c in range(8):
            off = r * 128 + c * 16
            so = pl.ds(off, 16)
            acc = dirs_v[pl.ds(off, 16)] * vals_v[pl.ds(off, 16)]
            for l in range(1, L):
                sl = pl.ds(l * NWK + off, 16)
                acc = acc + dirs_v[sl] * vals_v[sl]
            h = head_v[so]
            cs = cs_v[so]
            melt = melt_v[so]
            creep = creep_v[so]
            area = area_v[so]
            bnd = bnd_v[so]
            div = jnp.where(bnd > 0.5, h, acc / area)
            em = melt + 0.1 * div
            k1 = em - creep * cs
            k2 = em - creep * (cs + k1 * (DT / 2))
            k3 = em - creep * (cs + k2 * (DT / 2))
            k4 = em - creep * (cs + k3 * DT)
            out_v[so] = cs + (DT / 6.0) * (k1 + 2 * k2 + 2 * k3 + k4)
        return carry

    lax.fori_loop(0, NR, body, 0)
    pltpu.sync_copy(out_v, out_hbm.at[wid])


def kernel(hydraulic_head, conduit_size, transmissivity, melt_forcing,
           creep_closure, length_of_link, area_at_node,
           node_at_link_head, node_at_link_tail, links_at_node,
           link_dirs_at_node, node_is_boundary):
    ep = E_PAD - E
    hidx = jnp.pad(node_at_link_head, (0, ep)).reshape(NW, EW)
    tidx = jnp.pad(node_at_link_tail, (0, ep)).reshape(NW, EW)
    trans = jnp.pad(transmissivity, (0, ep)).reshape(NW, EW)
    lenl = jnp.pad(length_of_link, (0, ep), constant_values=1.0
                   ).reshape(NW, EW)

    np_ = N_PAD - N
    head_p = jnp.pad(hydraulic_head, (0, np_))
    cs_p = jnp.pad(conduit_size, (0, np_))

    flux = _flux_kernel(head_p, cs_p, trans, lenl, hidx, tidx)
    flux_flat = flux.reshape(E_PAD)

    linksT = jnp.pad(links_at_node, ((0, np_), (0, 0))).T
    dirsT = jnp.pad(link_dirs_at_node, ((0, np_), (0, 0))).T
    links_r = linksT.reshape(L, NW, NWK).transpose(1, 0, 2).reshape(NW, L * NWK)
    dirs_r = dirsT.astype(jnp.float32).reshape(L, NW, NWK
                                               ).transpose(1, 0, 2).reshape(NW, L * NWK)
    head_r = head_p.reshape(NW, NWK)
    cs_r = cs_p.reshape(NW, NWK)
    melt_r = jnp.pad(melt_forcing, (0, np_)).reshape(NW, NWK)
    creep_r = jnp.pad(creep_closure, (0, np_)).reshape(NW, NWK)
    area_r = jnp.pad(area_at_node, (0, np_), constant_values=1.0
                     ).reshape(NW, NWK)
    bnd_r = jnp.pad(node_is_boundary.astype(jnp.float32), (0, np_),
                    constant_values=1.0).reshape(NW, NWK)

    out = _node_kernel(flux_flat, links_r, dirs_r, head_r, cs_r, melt_r,
                       creep_r, area_r, bnd_r)
    return out.reshape(N_PAD)[:N]


# speedup vs baseline: 158.6726x; 1.0914x over previous
import functools

import jax
import jax.numpy as jnp
from jax import lax
from jax.experimental import pallas as pl
from jax.experimental.pallas import tpu as pltpu
from jax.experimental.pallas import tpu_sc as plsc

N = 100000
E = 400000
L = 8
DT = 0.1

NC = 2
NS = 16
NW = NC * NS

E_PAD = 409600
EW = E_PAD // NW
ER = EW // 128
N_PAD = 102400
NWK = N_PAD // NW
NR = NWK // 128

_mesh = plsc.VectorSubcoreMesh(core_axis_name="c", subcore_axis_name="s")


@functools.partial(
    pl.kernel,
    mesh=_mesh,
    out_type=jax.ShapeDtypeStruct((NW, EW), jnp.float32),
    scratch_types=[
        pltpu.VMEM_SHARED((N_PAD,), jnp.float32),
        pltpu.VMEM_SHARED((N_PAD,), jnp.float32),
        pltpu.VMEM((EW,), jnp.int32),
        pltpu.VMEM((EW,), jnp.int32),
        pltpu.VMEM((EW,), jnp.float32),
        pltpu.VMEM((EW,), jnp.float32),
        pltpu.VMEM((EW,), jnp.float32),
        pltpu.VMEM((EW,), jnp.float32),
        pltpu.VMEM((EW,), jnp.float32),
        pltpu.VMEM((EW,), jnp.float32),
        pltpu.VMEM((EW,), jnp.float32),
        pltpu.SemaphoreType.DMA,
    ],
)
def _flux_kernel(head_hbm, cs_hbm, trans_hbm, len_hbm, hidx_hbm, tidx_hbm,
                 out_hbm, head_sh, cs_sh, hidx_v, tidx_v, hh_v, ht_v, ch_v,
                 ct_v, tr_v, ln_v, fl_v, sem):
    sid = lax.axis_index("s")
    wid = sid * NC + lax.axis_index("c")
    seg = N_PAD // NS
    ss = pl.ds(sid * seg, seg)
    pltpu.async_copy(head_hbm.at[ss], head_sh.at[ss], sem)
    pltpu.async_copy(cs_hbm.at[ss], cs_sh.at[ss], sem)
    pltpu.async_copy(hidx_hbm.at[wid], hidx_v, sem)
    pltpu.async_copy(tidx_hbm.at[wid], tidx_v, sem)
    pltpu.async_copy(trans_hbm.at[wid], tr_v, sem)
    pltpu.async_copy(len_hbm.at[wid], ln_v, sem)
    pltpu.make_async_copy(head_hbm.at[ss], head_sh.at[ss], sem).wait()
    pltpu.make_async_copy(cs_hbm.at[ss], cs_sh.at[ss], sem).wait()
    pltpu.make_async_copy(hidx_hbm.at[wid], hidx_v, sem).wait()
    pltpu.make_async_copy(tidx_hbm.at[wid], tidx_v, sem).wait()
    pltpu.make_async_copy(trans_hbm.at[wid], tr_v, sem).wait()
    pltpu.make_async_copy(len_hbm.at[wid], ln_v, sem).wait()
    plsc.subcore_barrier()

    def fire(r, carry):
        sl = pl.ds(r * 128, 128)
        pltpu.async_copy(head_sh.at[hidx_v.at[sl]], hh_v.at[sl], sem)
        pltpu.async_copy(head_sh.at[tidx_v.at[sl]], ht_v.at[sl], sem)
        pltpu.async_copy(cs_sh.at[hidx_v.at[sl]], ch_v.at[sl], sem)
        pltpu.async_copy(cs_sh.at[tidx_v.at[sl]], ct_v.at[sl], sem)
        return carry

    lax.fori_loop(0, ER, fire, 0)
    for buf in (hh_v, ht_v, ch_v, ct_v):
        pltpu.make_async_copy(out_hbm.at[wid], buf, sem).wait()

    def body(r, carry):
        for c in range(8):
            s = pl.ds(r * 128 + c * 16, 16)
            fl_v[s] = ((-0.5) * tr_v[s] * (ch_v[s] + ct_v[s])
                       * ((hh_v[s] - ht_v[s]) / ln_v[s]))
        return carry

    lax.fori_loop(0, ER, body, 0)
    pltpu.sync_copy(fl_v, out_hbm.at[wid])


@functools.partial(
    pl.kernel,
    mesh=_mesh,
    out_type=jax.ShapeDtypeStruct((NW, NWK), jnp.float32),
    scratch_types=[
        pltpu.VMEM_SHARED((E_PAD,), jnp.float32),
        pltpu.VMEM((L * NWK,), jnp.int32),
        pltpu.VMEM((L * NWK,), jnp.float32),
        pltpu.VMEM((L * NWK,), jnp.float32),
        pltpu.VMEM((NWK,), jnp.float32),
        pltpu.VMEM((NWK,), jnp.float32),
        pltpu.VMEM((NWK,), jnp.float32),
        pltpu.VMEM((NWK,), jnp.float32),
        pltpu.VMEM((NWK,), jnp.float32),
        pltpu.VMEM((NWK,), jnp.float32),
        pltpu.VMEM((NWK,), jnp.float32),
        pltpu.SemaphoreType.DMA,
    ],
)
def _node_kernel(flux_hbm, links_hbm, dirs_hbm, head_hbm, cs_hbm, melt_hbm,
                 creep_hbm, area_hbm, bnd_hbm, out_hbm, flux_sh, lidx_v,
                 vals_v, dirs_v, head_v, cs_v, melt_v, creep_v, area_v,
                 bnd_v, out_v, sem):
    sid = lax.axis_index("s")
    wid = sid * NC + lax.axis_index("c")
    seg = E_PAD // NS
    ss = pl.ds(sid * seg, seg)
    pltpu.async_copy(flux_hbm.at[ss], flux_sh.at[ss], sem)
    pltpu.async_copy(links_hbm.at[wid], lidx_v, sem)
    pltpu.async_copy(dirs_hbm.at[wid], dirs_v, sem)
    pltpu.async_copy(head_hbm.at[wid], head_v, sem)
    pltpu.async_copy(cs_hbm.at[wid], cs_v, sem)
    pltpu.async_copy(melt_hbm.at[wid], melt_v, sem)
    pltpu.async_copy(creep_hbm.at[wid], creep_v, sem)
    pltpu.async_copy(area_hbm.at[wid], area_v, sem)
    pltpu.async_copy(bnd_hbm.at[wid], bnd_v, sem)
    pltpu.make_async_copy(flux_hbm.at[ss], flux_sh.at[ss], sem).wait()
    pltpu.make_async_copy(links_hbm.at[wid], lidx_v, sem).wait()
    pltpu.make_async_copy(dirs_hbm.at[wid], dirs_v, sem).wait()
    pltpu.make_async_copy(head_hbm.at[wid], head_v, sem).wait()
    pltpu.make_async_copy(cs_hbm.at[wid], cs_v, sem).wait()
    pltpu.make_async_copy(melt_hbm.at[wid], melt_v, sem).wait()
    pltpu.make_async_copy(creep_hbm.at[wid], creep_v, sem).wait()
    pltpu.make_async_copy(area_hbm.at[wid], area_v, sem).wait()
    pltpu.make_async_copy(bnd_hbm.at[wid], bnd_v, sem).wait()
    plsc.subcore_barrier()

    def fire(r, carry):
        for l in range(L):
            sl = pl.ds(l * NWK + r * 128, 128)
            pltpu.async_copy(flux_sh.at[lidx_v.at[sl]], vals_v.at[sl], sem)
        return carry

    lax.fori_loop(0, NR, fire, 0)
    pltpu.make_async_copy(dirs_hbm.at[wid], vals_v, sem).wait()

    def body(r, carry):
        for c in range(8):
            off = r * 128 + c * 16
            so = pl.ds(off, 16)
            acc = dirs_v[pl.ds(off, 16)] * vals_v[pl.ds(off, 16)]
            for l in range(1, L):
                sl = pl.ds(l * NWK + off, 16)
                acc = acc + dirs_v[sl] * vals_v[sl]
            h = head_v[so]
            cs = cs_v[so]
            melt = melt_v[so]
            creep = creep_v[so]
            area = area_v[so]
            bnd = bnd_v[so]
            div = jnp.where(bnd > 0.5, h, acc / area)
            em = melt + 0.1 * div
            k1 = em - creep * cs
            k2 = em - creep * (cs + k1 * (DT / 2))
            k3 = em - creep * (cs + k2 * (DT / 2))
            k4 = em - creep * (cs + k3 * DT)
            out_v[so] = cs + (DT / 6.0) * (k1 + 2 * k2 + 2 * k3 + k4)
        return carry

    lax.fori_loop(0, NR, body, 0)
    pltpu.sync_copy(out_v, out_hbm.at[wid])


def kernel(hydraulic_head, conduit_size, transmissivity, melt_forcing,
           creep_closure, length_of_link, area_at_node,
           node_at_link_head, node_at_link_tail, links_at_node,
           link_dirs_at_node, node_is_boundary):
    ep = E_PAD - E
    hidx = jnp.pad(node_at_link_head, (0, ep)).reshape(NW, EW)
    tidx = jnp.pad(node_at_link_tail, (0, ep)).reshape(NW, EW)
    trans = jnp.pad(transmissivity, (0, ep)).reshape(NW, EW)
    lenl = jnp.pad(length_of_link, (0, ep), constant_values=1.0
                   ).reshape(NW, EW)

    np_ = N_PAD - N
    head_p = jnp.pad(hydraulic_head, (0, np_))
    cs_p = jnp.pad(conduit_size, (0, np_))

    flux = _flux_kernel(head_p, cs_p, trans, lenl, hidx, tidx)
    flux_flat = flux.reshape(E_PAD)

    linksT = jnp.pad(links_at_node, ((0, np_), (0, 0))).T
    dirsT = jnp.pad(link_dirs_at_node, ((0, np_), (0, 0))).T
    links_r = linksT.reshape(L, NW, NWK).transpose(1, 0, 2).reshape(NW, L * NWK)
    dirs_r = dirsT.astype(jnp.float32).reshape(L, NW, NWK
                                               ).transpose(1, 0, 2).reshape(NW, L * NWK)
    head_r = head_p.reshape(NW, NWK)
    cs_r = cs_p.reshape(NW, NWK)
    melt_r = jnp.pad(melt_forcing, (0, np_)).reshape(NW, NWK)
    creep_r = jnp.pad(creep_closure, (0, np_)).reshape(NW, NWK)
    area_r = jnp.pad(area_at_node, (0, np_), constant_values=1.0
                     ).reshape(NW, NWK)
    bnd_r = jnp.pad(node_is_boundary.astype(jnp.float32), (0, np_),
                    constant_values=1.0).reshape(NW, NWK)

    out = _node_kernel(flux_flat, links_r, dirs_r, head_r, cs_r, melt_r,
                       creep_r, area_r, bnd_r)
    return out.reshape(N_PAD)[:N]


# speedup vs baseline: 160.5912x; 1.0121x over previous
import functools

import jax
import jax.numpy as jnp
from jax import lax
from jax.experimental import pallas as pl
from jax.experimental.pallas import tpu as pltpu
from jax.experimental.pallas import tpu_sc as plsc

N = 100000
E = 400000
L = 8
DT = 0.1

NC = 2
NS = 16
NW = NC * NS

E_PAD = 409600
EW = E_PAD // NW
ER = EW // 128
N_PAD = 102400
NWK = N_PAD // NW
NR = NWK // 128

_mesh = plsc.VectorSubcoreMesh(core_axis_name="c", subcore_axis_name="s")


@functools.partial(
    pl.kernel,
    mesh=_mesh,
    out_type=jax.ShapeDtypeStruct((NW, EW), jnp.float32),
    scratch_types=[
        pltpu.VMEM_SHARED((N_PAD,), jnp.float32),
        pltpu.VMEM_SHARED((N_PAD,), jnp.float32),
        pltpu.VMEM((EW,), jnp.int32),
        pltpu.VMEM((EW,), jnp.int32),
        pltpu.VMEM((EW,), jnp.float32),
        pltpu.VMEM((EW,), jnp.float32),
        pltpu.VMEM((EW,), jnp.float32),
        pltpu.VMEM((EW,), jnp.float32),
        pltpu.VMEM((EW,), jnp.float32),
        pltpu.VMEM((EW,), jnp.float32),
        pltpu.VMEM((EW,), jnp.float32),
        pltpu.SemaphoreType.DMA,
        pltpu.SemaphoreType.DMA,
    ],
)
def _flux_kernel(head_hbm, cs_hbm, trans_hbm, len_hbm, hidx_hbm, tidx_hbm,
                 out_hbm, head_sh, cs_sh, hidx_v, tidx_v, hh_v, ht_v, ch_v,
                 ct_v, tr_v, ln_v, fl_v, sem, sem2):
    sid = lax.axis_index("s")
    wid = sid * NC + lax.axis_index("c")
    seg = N_PAD // NS
    ss = pl.ds(sid * seg, seg)
    pltpu.async_copy(head_hbm.at[ss], head_sh.at[ss], sem)
    pltpu.async_copy(cs_hbm.at[ss], cs_sh.at[ss], sem)
    pltpu.async_copy(hidx_hbm.at[wid], hidx_v, sem)
    pltpu.async_copy(tidx_hbm.at[wid], tidx_v, sem)
    pltpu.async_copy(trans_hbm.at[wid], tr_v, sem)
    pltpu.async_copy(len_hbm.at[wid], ln_v, sem)
    pltpu.make_async_copy(head_hbm.at[ss], head_sh.at[ss], sem).wait()
    pltpu.make_async_copy(cs_hbm.at[ss], cs_sh.at[ss], sem).wait()
    pltpu.make_async_copy(hidx_hbm.at[wid], hidx_v, sem).wait()
    pltpu.make_async_copy(tidx_hbm.at[wid], tidx_v, sem).wait()
    pltpu.make_async_copy(trans_hbm.at[wid], tr_v, sem).wait()
    pltpu.make_async_copy(len_hbm.at[wid], ln_v, sem).wait()
    plsc.subcore_barrier()

    def fire_on(s_):
        def fire(r, carry):
            sl = pl.ds(r * 128, 128)
            pltpu.async_copy(head_sh.at[hidx_v.at[sl]], hh_v.at[sl], s_)
            pltpu.async_copy(head_sh.at[tidx_v.at[sl]], ht_v.at[sl], s_)
            pltpu.async_copy(cs_sh.at[hidx_v.at[sl]], ch_v.at[sl], s_)
            pltpu.async_copy(cs_sh.at[tidx_v.at[sl]], ct_v.at[sl], s_)
            return carry
        return fire

    EH = ER // 2
    lax.fori_loop(0, EH, fire_on(sem), 0)
    lax.fori_loop(EH, ER, fire_on(sem2), 0)

    def body(r, carry):
        for c in range(8):
            s = pl.ds(r * 128 + c * 16, 16)
            fl_v[s] = ((-0.5) * tr_v[s] * (ch_v[s] + ct_v[s])
                       * ((hh_v[s] - ht_v[s]) / ln_v[s]))
        return carry

    half = pl.ds(0, EH * 128)
    for buf in (hh_v, ht_v, ch_v, ct_v):
        pltpu.make_async_copy(out_hbm.at[wid, half], buf.at[half],
                              sem).wait()
    lax.fori_loop(0, EH, body, 0)
    for buf in (hh_v, ht_v, ch_v, ct_v):
        pltpu.make_async_copy(out_hbm.at[wid, half], buf.at[half],
                              sem2).wait()
    lax.fori_loop(EH, ER, body, 0)
    pltpu.sync_copy(fl_v, out_hbm.at[wid])


@functools.partial(
    pl.kernel,
    mesh=_mesh,
    out_type=jax.ShapeDtypeStruct((NW, NWK), jnp.float32),
    scratch_types=[
        pltpu.VMEM_SHARED((E_PAD,), jnp.float32),
        pltpu.VMEM((L * NWK,), jnp.int32),
        pltpu.VMEM((L * NWK,), jnp.float32),
        pltpu.VMEM((L * NWK,), jnp.float32),
        pltpu.VMEM((NWK,), jnp.float32),
        pltpu.VMEM((NWK,), jnp.float32),
        pltpu.VMEM((NWK,), jnp.float32),
        pltpu.VMEM((NWK,), jnp.float32),
        pltpu.VMEM((NWK,), jnp.float32),
        pltpu.VMEM((NWK,), jnp.float32),
        pltpu.VMEM((NWK,), jnp.float32),
        pltpu.SemaphoreType.DMA,
        pltpu.SemaphoreType.DMA,
    ],
)
def _node_kernel(flux_hbm, links_hbm, dirs_hbm, head_hbm, cs_hbm, melt_hbm,
                 creep_hbm, area_hbm, bnd_hbm, out_hbm, flux_sh, lidx_v,
                 vals_v, dirs_v, head_v, cs_v, melt_v, creep_v, area_v,
                 bnd_v, out_v, sem, sem2):
    sid = lax.axis_index("s")
    wid = sid * NC + lax.axis_index("c")
    seg = E_PAD // NS
    ss = pl.ds(sid * seg, seg)
    pltpu.async_copy(flux_hbm.at[ss], flux_sh.at[ss], sem)
    pltpu.async_copy(links_hbm.at[wid], lidx_v, sem)
    pltpu.async_copy(dirs_hbm.at[wid], dirs_v, sem)
    pltpu.async_copy(head_hbm.at[wid], head_v, sem)
    pltpu.async_copy(cs_hbm.at[wid], cs_v, sem)
    pltpu.async_copy(melt_hbm.at[wid], melt_v, sem)
    pltpu.async_copy(creep_hbm.at[wid], creep_v, sem)
    pltpu.async_copy(area_hbm.at[wid], area_v, sem)
    pltpu.async_copy(bnd_hbm.at[wid], bnd_v, sem)
    pltpu.make_async_copy(flux_hbm.at[ss], flux_sh.at[ss], sem).wait()
    pltpu.make_async_copy(links_hbm.at[wid], lidx_v, sem).wait()
    pltpu.make_async_copy(dirs_hbm.at[wid], dirs_v, sem).wait()
    pltpu.make_async_copy(head_hbm.at[wid], head_v, sem).wait()
    pltpu.make_async_copy(cs_hbm.at[wid], cs_v, sem).wait()
    pltpu.make_async_copy(melt_hbm.at[wid], melt_v, sem).wait()
    pltpu.make_async_copy(creep_hbm.at[wid], creep_v, sem).wait()
    pltpu.make_async_copy(area_hbm.at[wid], area_v, sem).wait()
    pltpu.make_async_copy(bnd_hbm.at[wid], bnd_v, sem).wait()
    plsc.subcore_barrier()

    def fire_on(s_):
        def fire(r, carry):
            for l in range(L):
                sl = pl.ds(l * NWK + r * 128, 128)
                pltpu.async_copy(flux_sh.at[lidx_v.at[sl]], vals_v.at[sl],
                                 s_)
            return carry
        return fire

    NH = NR // 2
    lax.fori_loop(0, NH, fire_on(sem), 0)
    lax.fori_loop(NH, NR, fire_on(sem2), 0)

    def body(r, carry):
        for c in range(8):
            off = r * 128 + c * 16
            so = pl.ds(off, 16)
            acc = dirs_v[pl.ds(off, 16)] * vals_v[pl.ds(off, 16)]
            for l in range(1, L):
                sl = pl.ds(l * NWK + off, 16)
                acc = acc + dirs_v[sl] * vals_v[sl]
            h = head_v[so]
            cs = cs_v[so]
            melt = melt_v[so]
            creep = creep_v[so]
            area = area_v[so]
            bnd = bnd_v[so]
            div = jnp.where(bnd > 0.5, h, acc / area)
            em = melt + 0.1 * div
            k1 = em - creep * cs
            k2 = em - creep * (cs + k1 * (DT / 2))
            k3 = em - creep * (cs + k2 * (DT / 2))
            k4 = em - creep * (cs + k3 * DT)
            out_v[so] = cs + (DT / 6.0) * (k1 + 2 * k2 + 2 * k3 + k4)
        return carry

    h0 = NH * L * 128
    pltpu.make_async_copy(flux_sh.at[pl.ds(0, h0)], vals_v.at[pl.ds(0, h0)],
                          sem).wait()
    lax.fori_loop(0, NH, body, 0)
    h1 = (NR - NH) * L * 128
    pltpu.make_async_copy(flux_sh.at[pl.ds(0, h1)], vals_v.at[pl.ds(0, h1)],
                          sem2).wait()
    lax.fori_loop(NH, NR, body, 0)
    pltpu.sync_copy(out_v, out_hbm.at[wid])


def kernel(hydraulic_head, conduit_size, transmissivity, melt_forcing,
           creep_closure, length_of_link, area_at_node,
           node_at_link_head, node_at_link_tail, links_at_node,
           link_dirs_at_node, node_is_boundary):
    ep = E_PAD - E
    hidx = jnp.pad(node_at_link_head, (0, ep)).reshape(NW, EW)
    tidx = jnp.pad(node_at_link_tail, (0, ep)).reshape(NW, EW)
    trans = jnp.pad(transmissivity, (0, ep)).reshape(NW, EW)
    lenl = jnp.pad(length_of_link, (0, ep), constant_values=1.0
                   ).reshape(NW, EW)

    np_ = N_PAD - N
    head_p = jnp.pad(hydraulic_head, (0, np_))
    cs_p = jnp.pad(conduit_size, (0, np_))

    flux = _flux_kernel(head_p, cs_p, trans, lenl, hidx, tidx)
    flux_flat = flux.reshape(E_PAD)

    linksT = jnp.pad(links_at_node, ((0, np_), (0, 0))).T
    dirsT = jnp.pad(link_dirs_at_node, ((0, np_), (0, 0))).T
    links_r = linksT.reshape(L, NW, NWK).transpose(1, 0, 2).reshape(NW, L * NWK)
    dirs_r = dirsT.astype(jnp.float32).reshape(L, NW, NWK
                                               ).transpose(1, 0, 2).reshape(NW, L * NWK)
    head_r = head_p.reshape(NW, NWK)
    cs_r = cs_p.reshape(NW, NWK)
    melt_r = jnp.pad(melt_forcing, (0, np_)).reshape(NW, NWK)
    creep_r = jnp.pad(creep_closure, (0, np_)).reshape(NW, NWK)
    area_r = jnp.pad(area_at_node, (0, np_), constant_values=1.0
                     ).reshape(NW, NWK)
    bnd_r = jnp.pad(node_is_boundary.astype(jnp.float32), (0, np_),
                    constant_values=1.0).reshape(NW, NWK)

    out = _node_kernel(flux_flat, links_r, dirs_r, head_r, cs_r, melt_r,
                       creep_r, area_r, bnd_r)
    return out.reshape(N_PAD)[:N]
